# baseline (speedup 1.0000x reference)
import functools

import jax
import jax.numpy as jnp
from jax import lax
from jax.experimental import pallas as pl
from jax.experimental.pallas import tpu as pltpu
from jax.experimental.pallas import tpu_sc as plsc

NC = 2
NS = 16
NW = NC * NS
K = 128
CW = 128
DUMP = 64
F32 = jnp.float32


CHW = 2
CROWS = 2 * CHW
NST = 3


def _sub_rows(n, s):
    big = -(-n // (NS * 8)) * 8
    start = min(s * big, n)
    stop = min((s + 1) * big, n)
    return start, stop - start


def _scatter_body(h_hbm, edges, zeros, out, acc, ibuf, stage, gsems, isems,
                  *, n, epad, wins):
    core = lax.axis_index("c")
    sub = lax.axis_index("s")
    wid = core * NS + sub
    nch = wins // CHW
    ebase = wid * wins * 2

    for q in range(NST):
        pltpu.async_copy(edges.at[pl.ds(ebase + q * CROWS, CROWS)],
                         ibuf.at[q], isems.at[q])

    for s in range(NS):
        start, cnt = _sub_rows(n, s)
        zc = min(max(epad - start, 0), cnt)

        @pl.when(jnp.logical_and(core == 0, sub == s))
        def _():
            pltpu.sync_copy(h_hbm.at[pl.ds(start, cnt)],
                            acc.at[pl.ds(start, cnt)])
            if zc > 0:
                pltpu.sync_copy(zeros.at[pl.ds(0, zc)],
                                acc.at[pl.ds(start, zc)])

        @pl.when(jnp.logical_and(core != 0, sub == s))
        def _():
            pltpu.sync_copy(zeros.at[pl.ds(0, cnt)],
                            acc.at[pl.ds(start, cnt)])

    plsc.subcore_barrier()

    pltpu.make_async_copy(edges.at[pl.ds(0, CROWS)], ibuf.at[0],
                          isems.at[0]).wait()
    pltpu.async_copy(h_hbm.at[ibuf.at[0, 0]], stage.at[0], gsems.at[0])
    pltpu.async_copy(h_hbm.at[ibuf.at[0, 2]], stage.at[1], gsems.at[1])
    pltpu.make_async_copy(edges.at[pl.ds(0, CROWS)], ibuf.at[1],
                          isems.at[1]).wait()
    pltpu.async_copy(h_hbm.at[ibuf.at[1, 0]], stage.at[2], gsems.at[2])

    def win(w, c):
        b = w % NST
        p = (w // CHW) % NST
        r = w % CHW
        pltpu.make_async_copy(h_hbm.at[pl.ds(0, K)], stage.at[b],
                              gsems.at[b]).wait()
        pltpu.sync_copy(stage.at[b], acc.at[ibuf.at[p, 2 * r + 1]], add=True)

        @pl.when(r == CHW - 1)
        def _():
            cjn = w // CHW + NST

            @pl.when(cjn < nch)
            def _():
                pltpu.async_copy(
                    edges.at[pl.ds(ebase + cjn * CROWS, CROWS)],
                    ibuf.at[p], isems.at[p])

        @pl.when(w + NST < wins)
        def _():
            wn = w + NST
            pn = (wn // CHW) % NST
            rn = wn % CHW

            @pl.when(rn == 0)
            def _():
                pltpu.make_async_copy(edges.at[pl.ds(0, CROWS)], ibuf.at[pn],
                                      isems.at[pn]).wait()
            pltpu.async_copy(h_hbm.at[ibuf.at[pn, 2 * rn]], stage.at[b],
                             gsems.at[b])
        return c
    lax.fori_loop(0, wins, win, 0)

    plsc.subcore_barrier()
    for s in range(NS):
        start, cnt = _sub_rows(n, s)

        @pl.when(sub == s)
        def _():
            pltpu.sync_copy(acc.at[pl.ds(start, cnt)],
                            out.at[core, pl.ds(start, cnt)])


def _make_scatter_kernel(n, emb, wins, epad):
    mesh = plsc.VectorSubcoreMesh(core_axis_name="c", subcore_axis_name="s")
    return pl.kernel(
        functools.partial(_scatter_body, n=n, epad=epad, wins=wins),
        out_type=jax.ShapeDtypeStruct((NC, n, emb), F32),
        mesh=mesh,
        scratch_types=[
            pltpu.VMEM_SHARED((n, emb), F32),
            pltpu.VMEM((NST, CROWS, K), jnp.int32),
            pltpu.VMEM((NST, K, emb), F32),
            pltpu.SemaphoreType.DMA((NST,)),
            pltpu.SemaphoreType.DMA((NST,)),
        ],
    )


def _cnt_body(elemix, out, acc, idxb, ones, zbuf, qsem, *, n, wins):
    core = lax.axis_index("c")
    sub = lax.axis_index("s")
    wid = core * NS + sub
    zn = zbuf.shape[0]

    pltpu.sync_copy(elemix.at[pl.ds(wid * wins, wins)], idxb)

    def zb(i, c):
        zbuf[pl.ds(i * 16, 16)] = jnp.zeros((16,), F32)
        return c
    lax.fori_loop(0, zn // 16, zb, 0)
    for q in range(8):
        ones[pl.ds(q * 16, 16)] = jnp.ones((16,), F32)

    words = (n * CW) // NS
    def zc(q, c):
        pltpu.sync_copy(zbuf, acc.at[pl.ds(sub * words + q * zn, zn)])
        return c
    lax.fori_loop(0, words // zn, zc, 0)
    plsc.subcore_barrier()

    def win(j, c):
        pltpu.async_copy(ones, acc.at[idxb.at[j]], qsem, add=True)

        @pl.when(j >= 16)
        def _():
            pltpu.make_async_copy(ones, acc.at[idxb.at[0]], qsem).wait()
        return c
    lax.fori_loop(0, wins, win, 0)

    def dr(j, c):
        pltpu.make_async_copy(ones, acc.at[idxb.at[0]], qsem).wait()
        return c
    lax.fori_loop(0, min(16, wins), dr, 0)

    plsc.subcore_barrier()
    pltpu.sync_copy(acc.at[pl.ds(sub * words, words)],
                    out.at[core, pl.ds(sub * words, words)])


def _make_cnt_kernel(n, wins):
    mesh = plsc.VectorSubcoreMesh(core_axis_name="c", subcore_axis_name="s")
    return pl.kernel(
        functools.partial(_cnt_body, n=n, wins=wins),
        out_type=jax.ShapeDtypeStruct((NC, n * CW), F32),
        mesh=mesh,
        scratch_types=[
            pltpu.VMEM_SHARED((n * CW,), F32),
            pltpu.VMEM((wins, K), jnp.int32),
            pltpu.VMEM((K,), F32),
            pltpu.VMEM((10000,), F32),
            pltpu.SemaphoreType.DMA,
        ],
    )


_PREC = None


def _emb_body(x0, x1, x2, tab, ca, cb, out, cnt):
    it = lax.broadcasted_iota(jnp.int32, (x0.shape[0], 128), 1)
    c0 = (x0[...] == it) & (it < 16)
    c1 = (x1[...] == it - 16) & (it >= 16) & (it < 32)
    c2 = (x2[...] == it - 32) & (it >= 32) & (it < 48)
    oh = jnp.where(c0 | c1 | c2, 1.0, 0.0)
    out[...] = jnp.dot(oh, tab[...], preferred_element_type=F32,
                       precision=_PREC)
    cnt[...] = ca[...] + cb[...]


def _emb_lookup(x, x_emb1, x_emb2, x_emb3, ca, cb, n, emb):
    tab = jnp.concatenate([
        jnp.pad(x_emb1, ((0, 16 - x_emb1.shape[0]), (0, 0))),
        jnp.pad(x_emb2, ((0, 16 - x_emb2.shape[0]), (0, 0))),
        jnp.pad(x_emb3, ((0, 16 - x_emb3.shape[0]), (0, 0))),
        jnp.zeros((80, emb), F32),
    ], axis=0)
    b = 1000
    grid = n // b
    return pl.pallas_call(
        _emb_body,
        grid=(grid,),
        in_specs=[
            pl.BlockSpec((b, 1), lambda i: (i, 0)),
            pl.BlockSpec((b, 1), lambda i: (i, 0)),
            pl.BlockSpec((b, 1), lambda i: (i, 0)),
            pl.BlockSpec((128, emb), lambda i: (0, 0)),
            pl.BlockSpec((b, CW), lambda i: (i, 0)),
            pl.BlockSpec((b, CW), lambda i: (i, 0)),
        ],
        out_specs=[pl.BlockSpec((b, emb), lambda i: (i, 0)),
                   pl.BlockSpec((b, CW), lambda i: (i, 0))],
        out_shape=[jax.ShapeDtypeStruct((n, emb), F32),
                   jax.ShapeDtypeStruct((n, CW), F32)],
    )(x[:, 0:1], x[:, 1:2], x[:, 2:3], tab, ca, cb)


def _mlp_body(p0, p1, cnt_r, ce, w1, b1, w2, b2, g, b, out, *, last):
    cnt = cnt_r[...]
    agg = p0[...] + p1[...] + jnp.dot(cnt, ce[...], preferred_element_type=F32,
                                      precision=_PREC)
    z = jnp.maximum(
        jnp.dot(agg, w1[...], preferred_element_type=F32, precision=_PREC)
        + b1[...], 0.0)
    h2 = (jnp.dot(z, w2[...], preferred_element_type=F32, precision=_PREC)
          + b2[...])
    mu = jnp.mean(h2, axis=-1, keepdims=True)
    var = jnp.mean((h2 - mu) ** 2, axis=-1, keepdims=True)
    hn = (h2 - mu) / jnp.sqrt(var + 1e-5) * g[...] + b[...]
    if not last:
        hn = jnp.maximum(hn, 0.0)
    out[...] = hn


def _mlp_layer(p0, p1, cnt, ce, w1, b1, w2, b2, g, b, *, last, n, emb):
    blk = 2000
    grid = n // blk
    full = lambda r, c: pl.BlockSpec((r, c), lambda i: (0, 0))
    row = lambda c: pl.BlockSpec((blk, c), lambda i: (i, 0))
    return pl.pallas_call(
        functools.partial(_mlp_body, last=last),
        grid=(grid,),
        in_specs=[
            row(emb), row(emb), row(CW),
            full(CW, emb), full(emb, 2 * emb), full(1, 2 * emb),
            full(2 * emb, emb), full(1, emb), full(1, emb), full(1, emb),
        ],
        out_specs=row(emb),
        out_shape=jax.ShapeDtypeStruct((n, emb), F32),
    )(p0, p1, cnt, ce, w1, b1[None], w2, b2[None], g[None], b[None])


def kernel(x, edge_index, edge_attr, batch, x_emb1, x_emb2, x_emb3,
           edge_emb1, edge_emb2, W1, b1, W2, b2, ln_g, ln_b):
    n = x.shape[0]
    e = edge_index.shape[1]
    emb = x_emb1.shape[1]
    nl = W1.shape[0]
    i32 = jnp.int32

    src = edge_index[0]
    dst = edge_index[1]
    combo = edge_attr[:, 0] * 3 + edge_attr[:, 1]

    wins = CHW * (-(-e // (NW * K * CHW)))
    e_pad = wins * NW * K
    epad = e_pad - e
    pad_sl = jnp.arange(epad, dtype=i32)
    src_p = jnp.concatenate([src, pad_sl]).reshape(NW * wins, K)
    dst_p = jnp.concatenate([dst, pad_sl]).reshape(NW * wins, K)
    edges_il = jnp.stack([src_p, dst_p], axis=1).reshape(NW * wins * 2, K)

    elem = dst * CW + combo
    elem_self = jnp.arange(n, dtype=i32) * CW + 12
    ne = e + n
    ewins = 8 * (-(-ne // (NW * K * 8)))
    nepad = ewins * NW * K - ne
    pad_elem = (jnp.arange(nepad, dtype=i32) % n) * CW + (CW - 1)
    elem_p = jnp.concatenate([elem, elem_self, pad_elem]).reshape(
        NW * ewins, K)

    ce = (edge_emb1[:, :, None, :] + edge_emb2[:, None, :, :]).reshape(
        nl, 18, emb)
    ce_pad = jnp.zeros((nl, CW, emb), F32).at[:, :18].set(ce)

    cnt2 = _make_cnt_kernel(n, ewins)(elem_p).reshape(NC, n, CW)
    h, cnt = _emb_lookup(x, x_emb1, x_emb2, x_emb3, cnt2[0], cnt2[1], n, emb)
    zeros = jnp.zeros((_sub_rows(n, 0)[1], emb), F32)
    scat = _make_scatter_kernel(n, emb, wins, epad)

    for l in range(nl):
        parts = scat(h, edges_il, zeros)
        h = _mlp_layer(parts[0], parts[1], cnt, ce_pad[l],
                       W1[l], b1[l], W2[l], b2[l], ln_g[l], ln_b[l],
                       last=(l == nl - 1), n=n, emb=emb)
    return h


# speedup vs baseline: 1.0391x; 1.0391x over previous
import functools

import jax
import jax.numpy as jnp
from jax import lax
from jax.experimental import pallas as pl
from jax.experimental.pallas import tpu as pltpu
from jax.experimental.pallas import tpu_sc as plsc

NC = 2
NS = 16
NW = NC * NS
K = 128
CW = 128
DUMP = 64
F32 = jnp.float32


CHW = 2
CROWS = 2 * CHW
NST = 3


def _sub_rows(n, s):
    big = -(-n // (NS * 8)) * 8
    start = min(s * big, n)
    stop = min((s + 1) * big, n)
    return start, stop - start


def _scatter_body(h_hbm, edges, zeros, out, acc, ibuf, stage, gsems, isems,
                  *, n, epad, wins):
    core = lax.axis_index("c")
    sub = lax.axis_index("s")
    wid = core * NS + sub
    nch = wins // CHW
    ebase = wid * wins * 2

    for q in range(NST):
        pltpu.async_copy(edges.at[pl.ds(ebase + q * CROWS, CROWS)],
                         ibuf.at[q], isems.at[q])

    for s in range(NS):
        start, cnt = _sub_rows(n, s)
        zc = min(max(epad - start, 0), cnt)

        @pl.when(jnp.logical_and(core == 0, sub == s))
        def _():
            pltpu.sync_copy(h_hbm.at[pl.ds(start, cnt)],
                            acc.at[pl.ds(start, cnt)])
            if zc > 0:
                pltpu.sync_copy(zeros.at[pl.ds(0, zc)],
                                acc.at[pl.ds(start, zc)])

        @pl.when(jnp.logical_and(core != 0, sub == s))
        def _():
            pltpu.sync_copy(zeros.at[pl.ds(0, cnt)],
                            acc.at[pl.ds(start, cnt)])

    plsc.subcore_barrier()

    pltpu.make_async_copy(edges.at[pl.ds(0, CROWS)], ibuf.at[0],
                          isems.at[0]).wait()
    pltpu.async_copy(h_hbm.at[ibuf.at[0, 0]], stage.at[0], gsems.at[0])
    pltpu.async_copy(h_hbm.at[ibuf.at[0, 2]], stage.at[1], gsems.at[1])
    pltpu.make_async_copy(edges.at[pl.ds(0, CROWS)], ibuf.at[1],
                          isems.at[1]).wait()
    pltpu.async_copy(h_hbm.at[ibuf.at[1, 0]], stage.at[2], gsems.at[2])

    def win(w, c):
        b = w % NST
        p = (w // CHW) % NST
        r = w % CHW
        pltpu.make_async_copy(h_hbm.at[pl.ds(0, K)], stage.at[b],
                              gsems.at[b]).wait()
        pltpu.sync_copy(stage.at[b], acc.at[ibuf.at[p, 2 * r + 1]], add=True)

        @pl.when(r == CHW - 1)
        def _():
            cjn = w // CHW + NST

            @pl.when(cjn < nch)
            def _():
                pltpu.async_copy(
                    edges.at[pl.ds(ebase + cjn * CROWS, CROWS)],
                    ibuf.at[p], isems.at[p])

        @pl.when(w + NST < wins)
        def _():
            wn = w + NST
            pn = (wn // CHW) % NST
            rn = wn % CHW

            @pl.when(rn == 0)
            def _():
                pltpu.make_async_copy(edges.at[pl.ds(0, CROWS)], ibuf.at[pn],
                                      isems.at[pn]).wait()
            pltpu.async_copy(h_hbm.at[ibuf.at[pn, 2 * rn]], stage.at[b],
                             gsems.at[b])
        return c
    lax.fori_loop(0, wins, win, 0)

    plsc.subcore_barrier()
    for s in range(NS):
        start, cnt = _sub_rows(n, s)

        @pl.when(sub == s)
        def _():
            pltpu.sync_copy(acc.at[pl.ds(start, cnt)],
                            out.at[core, pl.ds(start, cnt)])


def _make_scatter_kernel(n, emb, wins, epad):
    mesh = plsc.VectorSubcoreMesh(core_axis_name="c", subcore_axis_name="s")
    return pl.kernel(
        functools.partial(_scatter_body, n=n, epad=epad, wins=wins),
        out_type=jax.ShapeDtypeStruct((NC, n, emb), F32),
        mesh=mesh,
        scratch_types=[
            pltpu.VMEM_SHARED((n, emb), F32),
            pltpu.VMEM((NST, CROWS, K), jnp.int32),
            pltpu.VMEM((NST, K, emb), F32),
            pltpu.SemaphoreType.DMA((NST,)),
            pltpu.SemaphoreType.DMA((NST,)),
        ],
    )


def _cnt_body(elemix, out, acc, idxb, ones, zbuf, qsem, *, n, wins):
    core = lax.axis_index("c")
    sub = lax.axis_index("s")
    wid = core * NS + sub
    zn = zbuf.shape[0]

    pltpu.sync_copy(elemix.at[pl.ds(wid * wins, wins)], idxb)

    def zb(i, c):
        zbuf[pl.ds(i * 16, 16)] = jnp.zeros((16,), F32)
        return c
    lax.fori_loop(0, zn // 16, zb, 0)
    for q in range(8):
        ones[pl.ds(q * 16, 16)] = jnp.ones((16,), F32)

    words = (n * CW) // NS
    def zc(q, c):
        pltpu.sync_copy(zbuf, acc.at[pl.ds(sub * words + q * zn, zn)])
        return c
    lax.fori_loop(0, words // zn, zc, 0)
    plsc.subcore_barrier()

    def win(j, c):
        pltpu.async_copy(ones, acc.at[idxb.at[j]], qsem, add=True)

        @pl.when(j >= 8)
        def _():
            pltpu.make_async_copy(ones, acc.at[idxb.at[0]], qsem).wait()
        return c
    lax.fori_loop(0, wins, win, 0)

    def dr(j, c):
        pltpu.make_async_copy(ones, acc.at[idxb.at[0]], qsem).wait()
        return c
    lax.fori_loop(0, min(8, wins), dr, 0)

    plsc.subcore_barrier()
    pltpu.sync_copy(acc.at[pl.ds(sub * words, words)],
                    out.at[core, pl.ds(sub * words, words)])


def _make_cnt_kernel(n, wins):
    mesh = plsc.VectorSubcoreMesh(core_axis_name="c", subcore_axis_name="s")
    return pl.kernel(
        functools.partial(_cnt_body, n=n, wins=wins),
        out_type=jax.ShapeDtypeStruct((NC, n * CW), F32),
        mesh=mesh,
        scratch_types=[
            pltpu.VMEM_SHARED((n * CW,), F32),
            pltpu.VMEM((wins, K), jnp.int32),
            pltpu.VMEM((K,), F32),
            pltpu.VMEM((10000,), F32),
            pltpu.SemaphoreType.DMA,
        ],
    )


_PREC = None


def _emb_body(x0, x1, x2, tab, out):
    it = lax.broadcasted_iota(jnp.int32, (x0.shape[0], 128), 1)
    c0 = (x0[...] == it) & (it < 16)
    c1 = (x1[...] == it - 16) & (it >= 16) & (it < 32)
    c2 = (x2[...] == it - 32) & (it >= 32) & (it < 48)
    oh = jnp.where(c0 | c1 | c2, 1.0, 0.0)
    out[...] = jnp.dot(oh, tab[...], preferred_element_type=F32,
                       precision=_PREC)


def _emb_lookup(x, x_emb1, x_emb2, x_emb3, n, emb):
    tab = jnp.concatenate([
        jnp.pad(x_emb1, ((0, 16 - x_emb1.shape[0]), (0, 0))),
        jnp.pad(x_emb2, ((0, 16 - x_emb2.shape[0]), (0, 0))),
        jnp.pad(x_emb3, ((0, 16 - x_emb3.shape[0]), (0, 0))),
        jnp.zeros((80, emb), F32),
    ], axis=0)
    b = 1000
    grid = n // b
    return pl.pallas_call(
        _emb_body,
        grid=(grid,),
        in_specs=[
            pl.BlockSpec((b, 1), lambda i: (i, 0)),
            pl.BlockSpec((b, 1), lambda i: (i, 0)),
            pl.BlockSpec((b, 1), lambda i: (i, 0)),
            pl.BlockSpec((128, emb), lambda i: (0, 0)),
        ],
        out_specs=pl.BlockSpec((b, emb), lambda i: (i, 0)),
        out_shape=jax.ShapeDtypeStruct((n, emb), F32),
    )(x[:, 0:1], x[:, 1:2], x[:, 2:3], tab)


def _mlp_body(p0, p1, c0, c1, ce, w1, b1, w2, b2, g, b, out, *, last):
    cnt = c0[...] + c1[...]
    agg = p0[...] + p1[...] + jnp.dot(cnt, ce[...], preferred_element_type=F32,
                                      precision=_PREC)
    z = jnp.maximum(
        jnp.dot(agg, w1[...], preferred_element_type=F32, precision=_PREC)
        + b1[...], 0.0)
    h2 = (jnp.dot(z, w2[...], preferred_element_type=F32, precision=_PREC)
          + b2[...])
    mu = jnp.mean(h2, axis=-1, keepdims=True)
    var = jnp.mean((h2 - mu) ** 2, axis=-1, keepdims=True)
    hn = (h2 - mu) / jnp.sqrt(var + 1e-5) * g[...] + b[...]
    if not last:
        hn = jnp.maximum(hn, 0.0)
    out[...] = hn


def _mlp_layer(p0, p1, c0, c1, ce, w1, b1, w2, b2, g, b, *, last, n, emb):
    blk = 2000
    grid = n // blk
    full = lambda r, c: pl.BlockSpec((r, c), lambda i: (0, 0))
    row = lambda c: pl.BlockSpec((blk, c), lambda i: (i, 0))
    return pl.pallas_call(
        functools.partial(_mlp_body, last=last),
        grid=(grid,),
        in_specs=[
            row(emb), row(emb), row(CW), row(CW),
            full(CW, emb), full(emb, 2 * emb), full(1, 2 * emb),
            full(2 * emb, emb), full(1, emb), full(1, emb), full(1, emb),
        ],
        out_specs=row(emb),
        out_shape=jax.ShapeDtypeStruct((n, emb), F32),
    )(p0, p1, c0, c1, ce, w1, b1[None], w2, b2[None], g[None], b[None])


def kernel(x, edge_index, edge_attr, batch, x_emb1, x_emb2, x_emb3,
           edge_emb1, edge_emb2, W1, b1, W2, b2, ln_g, ln_b):
    n = x.shape[0]
    e = edge_index.shape[1]
    emb = x_emb1.shape[1]
    nl = W1.shape[0]
    i32 = jnp.int32

    src = edge_index[0]
    dst = edge_index[1]
    combo = edge_attr[:, 0] * 3 + edge_attr[:, 1]

    wins = CHW * (-(-e // (NW * K * CHW)))
    e_pad = wins * NW * K
    epad = e_pad - e
    pad_sl = jnp.arange(epad, dtype=i32)
    src_p = jnp.concatenate([src, pad_sl]).reshape(NW * wins, K)
    dst_p = jnp.concatenate([dst, pad_sl]).reshape(NW * wins, K)
    edges_il = jnp.stack([src_p, dst_p], axis=1).reshape(NW * wins * 2, K)

    elem = dst * CW + combo
    elem_self = jnp.arange(n, dtype=i32) * CW + 12
    ne = e + n
    ewins = 8 * (-(-ne // (NW * K * 8)))
    nepad = ewins * NW * K - ne
    pad_elem = (jnp.arange(nepad, dtype=i32) % n) * CW + (CW - 1)
    elem_p = jnp.concatenate([elem, elem_self, pad_elem]).reshape(
        NW * ewins, K)

    ce = (edge_emb1[:, :, None, :] + edge_emb2[:, None, :, :]).reshape(
        nl, 18, emb)
    ce_pad = jnp.zeros((nl, CW, emb), F32).at[:, :18].set(ce)

    h = _emb_lookup(x, x_emb1, x_emb2, x_emb3, n, emb)

    cnt2 = _make_cnt_kernel(n, ewins)(elem_p).reshape(NC, n, CW)
    zeros = jnp.zeros((_sub_rows(n, 0)[1], emb), F32)
    scat = _make_scatter_kernel(n, emb, wins, epad)

    for l in range(nl):
        parts = scat(h, edges_il, zeros)
        h = _mlp_layer(parts[0], parts[1], cnt2[0], cnt2[1], ce_pad[l],
                       W1[l], b1[l], W2[l], b2[l], ln_g[l], ln_b[l],
                       last=(l == nl - 1), n=n, emb=emb)
    return h
